# baseline (speedup 1.0000x reference)
import jax
import jax.numpy as jnp
from jax.experimental import pallas as pl
from jax.experimental.pallas import tpu as pltpu

_NSTEPS = 4


def _gmm_logprob_kernel(z_ref, mu_ref, log_var_ref, la_ref, out_ref,
                        ze_ref, m_ref, s_ref):
    i = pl.program_id(0)
    kk = mu_ref.shape[0]

    @pl.when(i == 0)
    def _():
        z = z_ref[...]
        ze_ref[...] = jnp.concatenate([z, z * z], axis=1)

    mu = mu_ref[...]
    log_var = log_var_ref[...]
    la_slice = la_ref[:, pl.ds(i * kk, kk)]

    var = jax.nn.softplus(log_var)
    inv_var = 1.0 / var
    log_det = jnp.sum(jnp.log(var), axis=1, keepdims=True)
    d = jnp.sum(mu * mu * inv_var, axis=1, keepdims=True)

    c = jnp.transpose(la_slice) - 0.5 * (log_det + d)
    w = jnp.concatenate([mu * inv_var, -0.5 * inv_var], axis=1)

    tt = jax.lax.dot_general(w, ze_ref[...], (((1,), (1,)), ((), ())),
                             preferred_element_type=jnp.float32)
    tt = tt + c
    m_i = jnp.max(tt, axis=0, keepdims=True)
    s_i = jnp.sum(jnp.exp(tt - m_i), axis=0, keepdims=True)

    @pl.when(i == 0)
    def _():
        m_ref[...] = m_i
        s_ref[...] = s_i

    @pl.when(i > 0)
    def _():
        m_old = m_ref[...]
        m_new = jnp.maximum(m_old, m_i)
        s_ref[...] = (s_ref[...] * jnp.exp(m_old - m_new)
                      + s_i * jnp.exp(m_i - m_new))
        m_ref[...] = m_new

    @pl.when(i == _NSTEPS - 1)
    def _():
        la_row = la_ref[...]
        la_max = jnp.max(la_row)
        log_norm = la_max + jnp.log(jnp.sum(jnp.exp(la_row - la_max)))
        out_ref[...] = m_ref[...] + (jnp.log(s_ref[...]) - log_norm)


def kernel(z, mu, log_var, log_alpha):
    B, F = z.shape
    K = mu.shape[0]
    KK = K // _NSTEPS
    out = pl.pallas_call(
        _gmm_logprob_kernel,
        grid=(_NSTEPS,),
        in_specs=[
            pl.BlockSpec((B, F), lambda i: (0, 0)),
            pl.BlockSpec((KK, F), lambda i: (i, 0)),
            pl.BlockSpec((KK, F), lambda i: (i, 0)),
            pl.BlockSpec((1, K), lambda i: (0, 0)),
        ],
        out_specs=pl.BlockSpec((1, B), lambda i: (0, 0)),
        out_shape=jax.ShapeDtypeStruct((1, B), jnp.float32),
        scratch_shapes=[
            pltpu.VMEM((B, 2 * F), jnp.float32),
            pltpu.VMEM((1, B), jnp.float32),
            pltpu.VMEM((1, B), jnp.float32),
        ],
    )(z, mu, log_var,
      jnp.concatenate([log_alpha, jnp.zeros((1,), log_alpha.dtype)]).reshape(1, K))
    return out.reshape(B)


# speedup vs baseline: 1.9758x; 1.9758x over previous
import jax
import jax.numpy as jnp
from jax.experimental import pallas as pl


def _gmm_logprob_kernel(z_ref, mu_ref, log_var_ref, la_ref, out_ref):
    z = z_ref[...]
    mu = mu_ref[...]
    log_var = log_var_ref[...]
    la_row = jnp.concatenate(
        [la_ref[...], jnp.zeros((1, 1), jnp.float32)], axis=1)

    var = jax.nn.softplus(log_var)
    inv_var = 1.0 / var
    log_det = jnp.sum(jnp.log(var), axis=1, keepdims=True)
    d = jnp.sum(mu * mu * inv_var, axis=1, keepdims=True)

    la_max = jnp.max(la_row)
    log_norm = la_max + jnp.log(jnp.sum(jnp.exp(la_row - la_max)))

    c = (jnp.transpose(la_row) - log_norm) - 0.5 * (log_det + d)
    w = jnp.concatenate([mu * inv_var, -0.5 * inv_var, c], axis=1)
    ze = jnp.concatenate(
        [z, z * z, jnp.ones((z.shape[0], 1), jnp.float32)], axis=1)

    tt = jax.lax.dot_general(w, ze, (((1,), (1,)), ((), ())),
                             preferred_element_type=jnp.float32)
    m = jnp.max(tt, axis=0, keepdims=True)
    out_ref[...] = m + jnp.log(
        jnp.sum(jnp.exp(tt - m), axis=0, keepdims=True))


def kernel(z, mu, log_var, log_alpha):
    B, F = z.shape
    K = mu.shape[0]
    out = pl.pallas_call(
        _gmm_logprob_kernel,
        grid=(1,),
        in_specs=[
            pl.BlockSpec((B, F), lambda i: (0, 0)),
            pl.BlockSpec((K, F), lambda i: (0, 0)),
            pl.BlockSpec((K, F), lambda i: (0, 0)),
            pl.BlockSpec((1, K - 1), lambda i: (0, 0)),
        ],
        out_specs=pl.BlockSpec((1, B), lambda i: (0, 0)),
        out_shape=jax.ShapeDtypeStruct((1, B), jnp.float32),
    )(z, mu, log_var, log_alpha.reshape(1, K - 1))
    return out.reshape(B)


# speedup vs baseline: 2.0056x; 1.0151x over previous
import jax
import jax.numpy as jnp
from jax.experimental import pallas as pl


def _gmm_logprob_kernel(z_ref, mu_ref, log_var_ref, la_ref, out_ref):
    z = z_ref[...]
    mu = mu_ref[...]
    log_var = log_var_ref[...]
    la_row = jnp.concatenate(
        [la_ref[...], jnp.zeros((1, 1), jnp.float32)], axis=1)

    var = jax.nn.softplus(log_var)
    inv_var = 1.0 / var
    log_det = jnp.sum(jnp.log(var), axis=1, keepdims=True)
    d = jnp.sum(mu * mu * inv_var, axis=1, keepdims=True)

    la_max = jnp.max(la_row)
    log_norm = la_max + jnp.log(jnp.sum(jnp.exp(la_row - la_max)))

    c = (jnp.transpose(la_row) - log_norm) - 0.5 * (log_det + d)
    w = jnp.concatenate([mu * inv_var, -0.5 * inv_var], axis=1)
    ze = jnp.concatenate([z, z * z], axis=1)

    tt = jax.lax.dot_general(w, ze, (((1,), (1,)), ((), ())),
                             preferred_element_type=jnp.float32)
    tt = tt + c
    m = jnp.max(tt, axis=0, keepdims=True)
    out_ref[...] = m + jnp.log(
        jnp.sum(jnp.exp(tt - m), axis=0, keepdims=True))


def kernel(z, mu, log_var, log_alpha):
    B, F = z.shape
    K = mu.shape[0]
    out = pl.pallas_call(
        _gmm_logprob_kernel,
        out_shape=jax.ShapeDtypeStruct((1, B), jnp.float32),
    )(z, mu, log_var, log_alpha.reshape(1, K - 1))
    return out.reshape(B)
